# baseline (speedup 1.0000x reference)
import functools

import jax
import jax.numpy as jnp
from jax import lax
from jax.experimental import pallas as pl
from jax.experimental.pallas import tpu as pltpu
from jax.experimental.pallas import tpu_sc as plsc

NUM_P = 100000
DIM = 16
BAT_N = 4096

TP = 2000


BC = BAT_N // 2


def _tc_body(hp_ref, hc_ref, xt_ref, val_ref, ind_ref, u_ref):
    j = pl.program_id(1)
    a = hp_ref[...]
    xt = xt_ref[...]
    u = lax.dot_general(a, xt, (((1,), (0,)), ((), ())),
                        preferred_element_type=jnp.float32)
    u_ref[...] = u + hc_ref[...]

    bv = u_ref[0:8, :]
    bi = jnp.zeros((8, BC), jnp.float32)
    for q in range(1, TP // 8):
        us = u_ref[q * 8:(q + 1) * 8, :]
        take = us > bv
        bv = jnp.maximum(us, bv)
        bi = jnp.where(take, jnp.float32(q), bi)

    s_iota = lax.broadcasted_iota(jnp.int32, (8, BC), 0)
    rloc = bi.astype(jnp.int32) * 8 + s_iota
    m = jnp.max(bv, axis=0)
    big = jnp.int32(2 ** 30)
    li = jnp.min(jnp.where(bv == m[None, :], rloc, big), axis=0)
    gi = li + j * TP

    @pl.when(j == 0)
    def _():
        val_ref[...] = m
        ind_ref[...] = gi

    @pl.when(j > 0)
    def _():
        pv = val_ref[...]
        take = m > pv
        val_ref[...] = jnp.maximum(m, pv)
        ind_ref[...] = jnp.where(take, gi, ind_ref[...])


def _tc_argmax(h_P, h, volP):
    nsteps = NUM_P // TP
    h2 = h.reshape(NUM_P, 1)
    xt = volP.T
    val, ind = pl.pallas_call(
        _tc_body,
        grid=(2, nsteps),
        in_specs=[
            pl.BlockSpec((TP, DIM), lambda c, j: (j, 0)),
            pl.BlockSpec((TP, 1), lambda c, j: (j, 0)),
            pl.BlockSpec((DIM, BC), lambda c, j: (0, c)),
        ],
        out_specs=[
            pl.BlockSpec((BC,), lambda c, j: (c,)),
            pl.BlockSpec((BC,), lambda c, j: (c,)),
        ],
        out_shape=[
            jax.ShapeDtypeStruct((BAT_N,), jnp.float32),
            jax.ShapeDtypeStruct((BAT_N,), jnp.int32),
        ],
        scratch_shapes=[pltpu.VMEM((TP, BC), jnp.float32)],
        compiler_params=pltpu.CompilerParams(
            dimension_semantics=("parallel", "arbitrary"),
        ),
    )(h_P, h2, xt)
    return val, ind


PAD_BINS = 102400
ZED = PAD_BINS // 16


def _sc_body(ind_hbm, g_hbm, idxs, upds, zed, acc):
    cid = lax.axis_index("c")
    sid = lax.axis_index("s")

    zeros16 = jnp.zeros((16,), jnp.float32)
    ones16 = jnp.full((16,), 1.0 / BAT_N, jnp.float32)

    def zloop(j, _):
        zed[pl.ds(j * 16, 16)] = zeros16
        return 0

    lax.fori_loop(0, ZED // 16, zloop, 0)

    def uloop(j, _):
        upds[pl.ds(j * 16, 16)] = ones16
        return 0

    lax.fori_loop(0, 256 // 16, uloop, 0)

    @pl.when(cid == 0)
    def _():
        pltpu.sync_copy(zed, acc.at[pl.ds(sid * ZED, ZED)])

    plsc.subcore_barrier()

    @pl.when(cid == 0)
    def _():
        pltpu.sync_copy(ind_hbm.at[pl.ds(sid * 2, 2)], idxs)
        pltpu.sync_copy(upds.at[pl.ds(0, 128)], acc.at[idxs.at[0]], add=True)
        pltpu.sync_copy(upds.at[pl.ds(128, 128)], acc.at[idxs.at[1]], add=True)

    plsc.subcore_barrier()

    @pl.when((cid == 0) & (sid == 0))
    def _():
        pltpu.sync_copy(acc, g_hbm)


@functools.partial(
    pl.kernel,
    out_type=jax.ShapeDtypeStruct((PAD_BINS,), jnp.float32),
    mesh=plsc.VectorSubcoreMesh(core_axis_name="c", subcore_axis_name="s"),
    scratch_types=[
        pltpu.VMEM((2, 128), jnp.int32),
        pltpu.VMEM((256,), jnp.float32),
        pltpu.VMEM((ZED,), jnp.float32),
        pltpu.VMEM_SHARED((PAD_BINS,), jnp.float32),
    ],
)
def _sc_hist(ind_hbm, g_hbm, idxs, upds, zed, acc):
    _sc_body(ind_hbm, g_hbm, idxs, upds, zed, acc)


def kernel(h_P, h, volP):
    val, ind = _tc_argmax(h_P, h, volP)
    gpad = _sc_hist(ind.reshape(32, 128))
    return val, gpad[:NUM_P]


# speedup vs baseline: 1.0318x; 1.0318x over previous
import functools

import jax
import jax.numpy as jnp
from jax import lax
from jax.experimental import pallas as pl
from jax.experimental.pallas import tpu as pltpu
from jax.experimental.pallas import tpu_sc as plsc

NUM_P = 100000
DIM = 16
BAT_N = 4096

TP = 2000


BC = BAT_N


def _tc_body(hp_ref, hc_ref, xt_ref, val_ref, ind_ref, u_ref):
    j = pl.program_id(0)
    a = hp_ref[...]
    xt = xt_ref[...]
    u = lax.dot_general(a, xt, (((1,), (0,)), ((), ())),
                        preferred_element_type=jnp.float32)
    u_ref[...] = u + hc_ref[...]

    bv = u_ref[0:8, :]
    bi = jnp.zeros((8, BC), jnp.float32)
    for q in range(1, TP // 8):
        us = u_ref[q * 8:(q + 1) * 8, :]
        take = us > bv
        bv = jnp.maximum(us, bv)
        bi = jnp.where(take, jnp.float32(q), bi)

    s_iota = lax.broadcasted_iota(jnp.int32, (8, BC), 0)
    rloc = bi.astype(jnp.int32) * 8 + s_iota
    m = jnp.max(bv, axis=0)
    big = jnp.int32(2 ** 30)
    li = jnp.min(jnp.where(bv == m[None, :], rloc, big), axis=0)
    gi = li + j * TP

    @pl.when(j == 0)
    def _():
        val_ref[...] = m
        ind_ref[...] = gi

    @pl.when(j > 0)
    def _():
        pv = val_ref[...]
        take = m > pv
        val_ref[...] = jnp.maximum(m, pv)
        ind_ref[...] = jnp.where(take, gi, ind_ref[...])


def _tc_argmax(h_P, h, volP):
    nsteps = NUM_P // TP
    h2 = h.reshape(NUM_P, 1)
    xt = volP.T
    val, ind = pl.pallas_call(
        _tc_body,
        grid=(nsteps,),
        in_specs=[
            pl.BlockSpec((TP, DIM), lambda j: (j, 0)),
            pl.BlockSpec((TP, 1), lambda j: (j, 0)),
            pl.BlockSpec((DIM, BC), lambda j: (0, 0)),
        ],
        out_specs=[
            pl.BlockSpec((BC,), lambda j: (0,)),
            pl.BlockSpec((BC,), lambda j: (0,)),
        ],
        out_shape=[
            jax.ShapeDtypeStruct((BAT_N,), jnp.float32),
            jax.ShapeDtypeStruct((BAT_N,), jnp.int32),
        ],
        scratch_shapes=[pltpu.VMEM((TP, BC), jnp.float32)],
        compiler_params=pltpu.CompilerParams(
            dimension_semantics=("arbitrary",),
        ),
    )(h_P, h2, xt)
    return val, ind


PAD_BINS = 102400
ZED = PAD_BINS // 16


def _sc_body(ind_hbm, g_hbm, idxs, upds, zed, acc):
    cid = lax.axis_index("c")
    sid = lax.axis_index("s")

    zeros16 = jnp.zeros((16,), jnp.float32)
    ones16 = jnp.full((16,), 1.0 / BAT_N, jnp.float32)

    def zloop(j, _):
        zed[pl.ds(j * 16, 16)] = zeros16
        return 0

    lax.fori_loop(0, ZED // 16, zloop, 0)

    def uloop(j, _):
        upds[pl.ds(j * 16, 16)] = ones16
        return 0

    lax.fori_loop(0, 256 // 16, uloop, 0)

    @pl.when(cid == 0)
    def _():
        pltpu.sync_copy(zed, acc.at[pl.ds(sid * ZED, ZED)])

    plsc.subcore_barrier()

    @pl.when(cid == 0)
    def _():
        pltpu.sync_copy(ind_hbm.at[pl.ds(sid * 2, 2)], idxs)
        pltpu.sync_copy(upds.at[pl.ds(0, 128)], acc.at[idxs.at[0]], add=True)
        pltpu.sync_copy(upds.at[pl.ds(128, 128)], acc.at[idxs.at[1]], add=True)

    plsc.subcore_barrier()

    @pl.when((cid == 0) & (sid == 0))
    def _():
        pltpu.sync_copy(acc, g_hbm)


@functools.partial(
    pl.kernel,
    out_type=jax.ShapeDtypeStruct((PAD_BINS,), jnp.float32),
    mesh=plsc.VectorSubcoreMesh(core_axis_name="c", subcore_axis_name="s"),
    scratch_types=[
        pltpu.VMEM((2, 128), jnp.int32),
        pltpu.VMEM((256,), jnp.float32),
        pltpu.VMEM((ZED,), jnp.float32),
        pltpu.VMEM_SHARED((PAD_BINS,), jnp.float32),
    ],
)
def _sc_hist(ind_hbm, g_hbm, idxs, upds, zed, acc):
    _sc_body(ind_hbm, g_hbm, idxs, upds, zed, acc)


def kernel(h_P, h, volP):
    val, ind = _tc_argmax(h_P, h, volP)
    gpad = _sc_hist(ind.reshape(32, 128))
    return val, gpad[:NUM_P]


# speedup vs baseline: 1.0570x; 1.0244x over previous
import functools

import jax
import jax.numpy as jnp
from jax import lax
from jax.experimental import pallas as pl
from jax.experimental.pallas import tpu as pltpu
from jax.experimental.pallas import tpu_sc as plsc

NUM_P = 100000
DIM = 16
BAT_N = 4096

TP = 4000


BC = BAT_N


def _tc_body(hp_ref, hc_ref, xt_ref, val_ref, ind_ref, u_ref):
    j = pl.program_id(0)
    a = hp_ref[...]
    xt = xt_ref[...]
    u = lax.dot_general(a, xt, (((1,), (0,)), ((), ())),
                        preferred_element_type=jnp.float32)
    u_ref[...] = u + hc_ref[...]

    bv = u_ref[0:8, :]
    bi = jnp.zeros((8, BC), jnp.float32)
    for q in range(1, TP // 8):
        us = u_ref[q * 8:(q + 1) * 8, :]
        take = us > bv
        bv = jnp.maximum(us, bv)
        bi = jnp.where(take, jnp.float32(q), bi)

    s_iota = lax.broadcasted_iota(jnp.int32, (8, BC), 0)
    rloc = bi.astype(jnp.int32) * 8 + s_iota
    m = jnp.max(bv, axis=0)
    big = jnp.int32(2 ** 30)
    li = jnp.min(jnp.where(bv == m[None, :], rloc, big), axis=0)
    gi = li + j * TP

    @pl.when(j == 0)
    def _():
        val_ref[...] = m
        ind_ref[...] = gi

    @pl.when(j > 0)
    def _():
        pv = val_ref[...]
        take = m > pv
        val_ref[...] = jnp.maximum(m, pv)
        ind_ref[...] = jnp.where(take, gi, ind_ref[...])


def _tc_argmax(h_P, h, volP):
    nsteps = NUM_P // TP
    h2 = h.reshape(NUM_P, 1)
    xt = volP.T
    val, ind = pl.pallas_call(
        _tc_body,
        grid=(nsteps,),
        in_specs=[
            pl.BlockSpec((TP, DIM), lambda j: (j, 0)),
            pl.BlockSpec((TP, 1), lambda j: (j, 0)),
            pl.BlockSpec((DIM, BC), lambda j: (0, 0)),
        ],
        out_specs=[
            pl.BlockSpec((BC,), lambda j: (0,)),
            pl.BlockSpec((BC,), lambda j: (0,)),
        ],
        out_shape=[
            jax.ShapeDtypeStruct((BAT_N,), jnp.float32),
            jax.ShapeDtypeStruct((BAT_N,), jnp.int32),
        ],
        scratch_shapes=[pltpu.VMEM((TP, BC), jnp.float32)],
        compiler_params=pltpu.CompilerParams(
            dimension_semantics=("arbitrary",),
        ),
    )(h_P, h2, xt)
    return val, ind


PAD_BINS = 102400
ZED = PAD_BINS // 16


def _sc_body(ind_hbm, g_hbm, idxs, upds, zed, acc):
    cid = lax.axis_index("c")
    sid = lax.axis_index("s")

    zeros16 = jnp.zeros((16,), jnp.float32)
    ones16 = jnp.full((16,), 1.0 / BAT_N, jnp.float32)

    def zloop(j, _):
        zed[pl.ds(j * 16, 16)] = zeros16
        return 0

    lax.fori_loop(0, ZED // 16, zloop, 0)

    def uloop(j, _):
        upds[pl.ds(j * 16, 16)] = ones16
        return 0

    lax.fori_loop(0, 256 // 16, uloop, 0)

    @pl.when(cid == 0)
    def _():
        pltpu.sync_copy(zed, acc.at[pl.ds(sid * ZED, ZED)])

    plsc.subcore_barrier()

    @pl.when(cid == 0)
    def _():
        pltpu.sync_copy(ind_hbm.at[pl.ds(sid * 2, 2)], idxs)
        pltpu.sync_copy(upds.at[pl.ds(0, 128)], acc.at[idxs.at[0]], add=True)
        pltpu.sync_copy(upds.at[pl.ds(128, 128)], acc.at[idxs.at[1]], add=True)

    plsc.subcore_barrier()

    @pl.when((cid == 0) & (sid == 0))
    def _():
        pltpu.sync_copy(acc, g_hbm)


@functools.partial(
    pl.kernel,
    out_type=jax.ShapeDtypeStruct((PAD_BINS,), jnp.float32),
    mesh=plsc.VectorSubcoreMesh(core_axis_name="c", subcore_axis_name="s"),
    scratch_types=[
        pltpu.VMEM((2, 128), jnp.int32),
        pltpu.VMEM((256,), jnp.float32),
        pltpu.VMEM((ZED,), jnp.float32),
        pltpu.VMEM_SHARED((PAD_BINS,), jnp.float32),
    ],
)
def _sc_hist(ind_hbm, g_hbm, idxs, upds, zed, acc):
    _sc_body(ind_hbm, g_hbm, idxs, upds, zed, acc)


def kernel(h_P, h, volP):
    val, ind = _tc_argmax(h_P, h, volP)
    gpad = _sc_hist(ind.reshape(32, 128))
    return val, gpad[:NUM_P]
